# baseline (speedup 1.0000x reference)
import functools

import jax
import jax.numpy as jnp
from jax import lax
from jax.experimental import pallas as pl
from jax.experimental.pallas import tpu as pltpu
from jax.experimental.pallas import tpu_sc as plsc

N = 10000
E = 320000
D = 128
NC = 2
NS = 16
NW = NC * NS
EPT = E // NW
CH = 128
EPT_PAD = 10240
NCHUNK = EPT_PAD // CH
N_PAD = 10240
RPT = N_PAD // NS


def _sum_body(with_cnt, x_hbm, srcg, dstg, zrow, zcnt, ones_hbm, *refs):
    if with_cnt:
        (sums_out, cnt_out, sidx, didx, rows0, rows1, ones,
         acc, cntacc, isem0, isem1, gsem0, gsem1, csem) = refs
    else:
        (sums_out, sidx, didx, rows0, rows1,
         acc, isem0, isem1, gsem0, gsem1) = refs
    isem = (isem0, isem1)
    gsem = (gsem0, gsem1)
    rows = (rows0, rows1)
    cid = lax.axis_index("c")
    sid = lax.axis_index("s")
    wid = cid * NS + sid

    def fetch_idx(j, p):
        pltpu.async_copy(srcg.at[wid].at[j], sidx.at[p], isem[p])
        pltpu.async_copy(dstg.at[wid].at[j], didx.at[p], isem[p])

    def wait_idx(p):
        pltpu.make_async_copy(srcg.at[wid].at[0], sidx.at[p],
                              isem[p]).wait()
        pltpu.make_async_copy(dstg.at[wid].at[0], didx.at[p],
                              isem[p]).wait()

    def gather(p):
        pltpu.async_copy(x_hbm.at[sidx.at[p]], rows[p], gsem[p])

    def wait_gather(p):
        pltpu.make_async_copy(x_hbm.at[sidx.at[0]], rows[p],
                              gsem[p]).wait()

    def drain_cnt():
        pltpu.make_async_copy(ones, cntacc.at[didx.at[0]], csem).wait()

    fetch_idx(0, 0)
    fetch_idx(1, 1)
    wait_idx(0)
    gather(0)
    pltpu.sync_copy(zrow, acc.at[pl.ds(sid * RPT, RPT)])
    if with_cnt:
        pltpu.sync_copy(zcnt, cntacc.at[pl.ds(sid * RPT, RPT)])
        pltpu.sync_copy(ones_hbm, ones)
    plsc.subcore_barrier()

    def chunk_step(c, p, prefetch, issue_next=True):
        q = 1 - p
        if issue_next:
            wait_idx(q)
            gather(q)
        wait_gather(p)
        if with_cnt:
            pltpu.async_copy(ones, cntacc.at[didx.at[p]], csem, add=True)
        pltpu.sync_copy(rows[p], acc.at[didx.at[p]], add=True)
        if with_cnt:
            drain_cnt()
        if prefetch:
            fetch_idx(c + 2, p)

    def pair_body(t, _):
        j = 2 * t
        chunk_step(j, 0, prefetch=True)
        chunk_step(j + 1, 1, prefetch=True)
        return 0

    lax.fori_loop(0, NCHUNK // 2 - 1, pair_body, 0)
    chunk_step(NCHUNK - 2, 0, prefetch=False)
    chunk_step(NCHUNK - 1, 1, prefetch=False, issue_next=False)
    plsc.subcore_barrier()

    pltpu.sync_copy(acc.at[pl.ds(sid * RPT, RPT)],
                    sums_out.at[cid].at[pl.ds(sid * RPT, RPT)])
    if with_cnt:
        pltpu.sync_copy(cntacc.at[pl.ds(sid * RPT, RPT)],
                        cnt_out.at[cid].at[pl.ds(sid * RPT, RPT)])


_MESH = plsc.VectorSubcoreMesh(core_axis_name="c", subcore_axis_name="s")


def _make_agg(with_cnt):
    out_type = [jax.ShapeDtypeStruct((NC, N_PAD, D), jnp.float32)]
    scratch = [
        pltpu.VMEM((2, CH), jnp.int32),
        pltpu.VMEM((2, CH), jnp.int32),
        pltpu.VMEM((CH, D), jnp.float32),
        pltpu.VMEM((CH, D), jnp.float32),
    ]
    if with_cnt:
        out_type.append(jax.ShapeDtypeStruct((NC, N_PAD), jnp.float32))
        scratch.append(pltpu.VMEM((CH,), jnp.float32))
    scratch.append(pltpu.VMEM_SHARED((N_PAD, D), jnp.float32))
    if with_cnt:
        scratch.append(pltpu.VMEM_SHARED((N_PAD,), jnp.float32))
    scratch += [pltpu.SemaphoreType.DMA] * 4
    if with_cnt:
        scratch.append(pltpu.SemaphoreType.DMA)
    return pl.kernel(
        functools.partial(_sum_body, with_cnt),
        out_type=out_type,
        mesh=_MESH,
        scratch_types=scratch,
        name=f"sage_agg{'_cnt' if with_cnt else ''}",
    )


_agg_with_cnt = _make_agg(True)
_agg_no_cnt = _make_agg(False)


def _layer_tc_body(s_ref, c_ref, x_ref, wl_ref, bl_ref, wr_ref, o_ref):
    s = s_ref[0] + s_ref[1]
    c = (c_ref[0, 0, :] + c_ref[0, 1, :])[:, None]
    mean = s / jnp.maximum(c, 1.0)
    acc = jnp.dot(mean, wl_ref[...], preferred_element_type=jnp.float32)
    acc += jnp.dot(x_ref[...], wr_ref[...], preferred_element_type=jnp.float32)
    o_ref[...] = jnp.maximum(acc + bl_ref[...], 0.0)


_TCB = 1024


def _layer_tc(sums, cnts, x, Wlt, bl, Wrt):
    return pl.pallas_call(
        _layer_tc_body,
        grid=(N_PAD // _TCB,),
        in_specs=[
            pl.BlockSpec((NC, _TCB, D), lambda i: (0, i, 0)),
            pl.BlockSpec((1, NC, _TCB), lambda i: (i, 0, 0)),
            pl.BlockSpec((_TCB, D), lambda i: (i, 0)),
            pl.BlockSpec((D, D), lambda i: (0, 0)),
            pl.BlockSpec((1, D), lambda i: (0, 0)),
            pl.BlockSpec((D, D), lambda i: (0, 0)),
        ],
        out_specs=pl.BlockSpec((_TCB, D), lambda i: (i, 0)),
        out_shape=jax.ShapeDtypeStruct((N, D), jnp.float32),
    )(sums, cnts, x, Wlt, bl.reshape(1, D), Wrt)


def kernel(x, edge_index, W1l, b1l, W1r, W2l, b2l, W2r):
    src = jnp.pad(edge_index[0].reshape(NW, EPT),
                  ((0, 0), (0, EPT_PAD - EPT)))
    dst = jnp.pad(edge_index[1].reshape(NW, EPT),
                  ((0, 0), (0, EPT_PAD - EPT)),
                  constant_values=N_PAD - 1)
    src = src.reshape(NW, NCHUNK, CH)
    dst = dst.reshape(NW, NCHUNK, CH)
    zrow = jnp.zeros((RPT, D), jnp.float32)
    zcnt = jnp.zeros((RPT,), jnp.float32)
    ones = jnp.ones((CH,), jnp.float32)

    sums1, cnt = _agg_with_cnt(x, src, dst, zrow, zcnt, ones)
    cnt_b = cnt.reshape(NC, N_PAD // _TCB, _TCB).transpose(1, 0, 2)
    h = _layer_tc(sums1, cnt_b, x, W1l.T, b1l, W1r.T)
    (sums2,) = _agg_no_cnt(h, src, dst, zrow, zcnt, ones)
    out = _layer_tc(sums2, cnt_b, h, W2l.T, b2l, W2r.T)
    return out


# speedup vs baseline: 2.9194x; 2.9194x over previous
import functools

import jax
import jax.numpy as jnp
from jax import lax
from jax.experimental import pallas as pl
from jax.experimental.pallas import tpu as pltpu
from jax.experimental.pallas import tpu_sc as plsc

N = 10000
E = 320000
D = 128
NC = 2
NS = 16
NW = NC * NS
EPT = E // NW
CH = 80
NCHUNK = EPT // CH
N_PAD = 10240
RPT = N_PAD // NS


def _sum_body(with_cnt, x_hbm, srcg, dstg, zrow, zcnt, ones_hbm, *refs):
    if with_cnt:
        (sums_out, cnt_out, src_idx, dst_idx, rows0, rows1, ones,
         acc, cntacc, sem0, sem1, csem) = refs
    else:
        (sums_out, src_idx, dst_idx, rows0, rows1,
         acc, sem0, sem1) = refs
    cid = lax.axis_index("c")
    sid = lax.axis_index("s")
    wid = cid * NS + sid

    pltpu.sync_copy(zrow, acc.at[pl.ds(sid * RPT, RPT)])
    if with_cnt:
        pltpu.sync_copy(zcnt, cntacc.at[pl.ds(sid * RPT, RPT)])
        pltpu.sync_copy(ones_hbm, ones)
    pltpu.sync_copy(srcg.at[wid], src_idx)
    pltpu.sync_copy(dstg.at[wid], dst_idx)
    plsc.subcore_barrier()

    def gather(j, buf, sem):
        return pltpu.async_copy(
            x_hbm.at[src_idx.at[pl.ds(j * CH, CH)]], buf, sem)

    def scatter(j, buf):
        if with_cnt:
            pltpu.async_copy(ones, cntacc.at[dst_idx.at[j]], csem,
                             add=True)
        pltpu.sync_copy(buf, acc.at[dst_idx.at[j]], add=True)

    def wait(buf, sem):
        pltpu.make_async_copy(x_hbm.at[src_idx.at[pl.ds(0, CH)]],
                              buf, sem).wait()

    gather(0, rows0, sem0)

    def pair_body(t, _):
        j = 2 * t
        gather(j + 1, rows1, sem1)
        wait(rows0, sem0)
        scatter(j, rows0)

        @pl.when(j + 2 < NCHUNK)
        def _():
            gather(j + 2, rows0, sem0)

        wait(rows1, sem1)
        scatter(j + 1, rows1)
        return 0

    lax.fori_loop(0, NCHUNK // 2, pair_body, 0)
    wait(rows0, sem0)
    scatter(NCHUNK - 1, rows0)

    if with_cnt:
        def drain(j, _):
            pltpu.make_async_copy(ones, cntacc.at[dst_idx.at[0]],
                                  csem).wait()
            return 0
        lax.fori_loop(0, NCHUNK, drain, 0)
    plsc.subcore_barrier()

    pltpu.sync_copy(acc.at[pl.ds(sid * RPT, RPT)],
                    sums_out.at[cid].at[pl.ds(sid * RPT, RPT)])
    if with_cnt:
        pltpu.sync_copy(cntacc.at[pl.ds(sid * RPT, RPT)],
                        cnt_out.at[cid].at[pl.ds(sid * RPT, RPT)])


_MESH = plsc.VectorSubcoreMesh(core_axis_name="c", subcore_axis_name="s")


def _make_agg(with_cnt):
    out_type = [jax.ShapeDtypeStruct((NC, N_PAD, D), jnp.float32)]
    scratch = [
        pltpu.VMEM((EPT,), jnp.int32),
        pltpu.VMEM((NCHUNK, CH), jnp.int32),
        pltpu.VMEM((CH, D), jnp.float32),
        pltpu.VMEM((CH, D), jnp.float32),
    ]
    if with_cnt:
        out_type.append(jax.ShapeDtypeStruct((NC, N_PAD), jnp.float32))
        scratch.append(pltpu.VMEM((CH,), jnp.float32))
    scratch.append(pltpu.VMEM_SHARED((N_PAD, D), jnp.float32))
    if with_cnt:
        scratch.append(pltpu.VMEM_SHARED((N_PAD,), jnp.float32))
    scratch += [pltpu.SemaphoreType.DMA, pltpu.SemaphoreType.DMA]
    if with_cnt:
        scratch.append(pltpu.SemaphoreType.DMA)
    return pl.kernel(
        functools.partial(_sum_body, with_cnt),
        out_type=out_type,
        mesh=_MESH,
        scratch_types=scratch,
        name=f"sage_agg{'_cnt' if with_cnt else ''}",
    )


_agg_with_cnt = _make_agg(True)
_agg_no_cnt = _make_agg(False)


def _layer_tc_body(s_ref, c_ref, x_ref, wl_ref, bl_ref, wr_ref, o_ref):
    s = s_ref[0] + s_ref[1]
    c = (c_ref[0, 0, :] + c_ref[0, 1, :])[:, None]
    mean = s / jnp.maximum(c, 1.0)
    acc = jnp.dot(mean, wl_ref[...], preferred_element_type=jnp.float32)
    acc += jnp.dot(x_ref[...], wr_ref[...], preferred_element_type=jnp.float32)
    o_ref[...] = jnp.maximum(acc + bl_ref[...], 0.0)


_TCB = 1024


def _layer_tc(sums, cnts, x, Wlt, bl, Wrt):
    return pl.pallas_call(
        _layer_tc_body,
        grid=(N_PAD // _TCB,),
        in_specs=[
            pl.BlockSpec((NC, _TCB, D), lambda i: (0, i, 0)),
            pl.BlockSpec((1, NC, _TCB), lambda i: (i, 0, 0)),
            pl.BlockSpec((_TCB, D), lambda i: (i, 0)),
            pl.BlockSpec((D, D), lambda i: (0, 0)),
            pl.BlockSpec((1, D), lambda i: (0, 0)),
            pl.BlockSpec((D, D), lambda i: (0, 0)),
        ],
        out_specs=pl.BlockSpec((_TCB, D), lambda i: (i, 0)),
        out_shape=jax.ShapeDtypeStruct((N, D), jnp.float32),
    )(sums, cnts, x, Wlt, bl.reshape(1, D), Wrt)


def kernel(x, edge_index, W1l, b1l, W1r, W2l, b2l, W2r):
    src = edge_index[0].reshape(NW, EPT)
    dst = edge_index[1].reshape(NW, NCHUNK, CH)
    zrow = jnp.zeros((RPT, D), jnp.float32)
    zcnt = jnp.zeros((RPT,), jnp.float32)
    ones = jnp.ones((CH,), jnp.float32)

    sums1, cnt = _agg_with_cnt(x, src, dst, zrow, zcnt, ones)
    cnt_b = cnt.reshape(NC, N_PAD // _TCB, _TCB).transpose(1, 0, 2)
    h = _layer_tc(sums1, cnt_b, x, W1l.T, b1l, W1r.T)
    (sums2,) = _agg_no_cnt(h, src, dst, zrow, zcnt, ones)
    out = _layer_tc(sums2, cnt_b, h, W2l.T, b2l, W2r.T)
    return out
